# baseline (speedup 1.0000x reference)
import functools

import jax
import jax.numpy as jnp
from jax import lax
from jax.experimental import pallas as pl
from jax.experimental.pallas import tpu as pltpu
from jax.experimental.pallas import tpu_sc as plsc

_N = 10000
_E = 320000
_P = 1280000
_D = 128
_L = 3
_Z = 1800

_NC = 2
_NS = 16
_NW = _NC * _NS

_CA = 256
_NCA = 1280
_NKA = _NCA // _NW
_NSTARTS = 1336
_CB = 64
_NK = 158
_PADE = _NW * _NK * _CB
_ROWS_PT = _N // _NS

def _sc_mesh():
    return plsc.VectorSubcoreMesh(core_axis_name="c", subcore_axis_name="s",
                                  num_cores=_NC, num_subcores=_NS)


def _zero_rows(buf, nrows):
    zv = jnp.zeros((16,), jnp.float32)

    def body(i, carry):
        for j in range(_D // 16):
            buf[i, pl.ds(16 * j, 16)] = zv
        return carry

    lax.fori_loop(0, nrows, body, 0)


@functools.lru_cache(maxsize=None)
def _build_z0_kernel():
    return functools.partial(
        pl.kernel,
        out_type=jax.ShapeDtypeStruct((_E, _D), jnp.float32),
        mesh=_sc_mesh(),
        scratch_types=[
            pltpu.VMEM((56,), jnp.int32),
            pltpu.VMEM((128,), jnp.float32), pltpu.VMEM((128,), jnp.float32),
            pltpu.VMEM((128,), jnp.float32),
            pltpu.VMEM((128,), jnp.int32), pltpu.VMEM((128,), jnp.int32),
            pltpu.VMEM((128,), jnp.int32),
            pltpu.VMEM((128,), jnp.int32), pltpu.VMEM((128,), jnp.int32),
            pltpu.VMEM((128,), jnp.int32),
            pltpu.VMEM((128,), jnp.int32), pltpu.VMEM((128,), jnp.int32),
            pltpu.VMEM((128,), jnp.int32),
            pltpu.VMEM((128, _D), jnp.float32),
            pltpu.VMEM((128, _D), jnp.float32),
            pltpu.VMEM((128, _D), jnp.float32),
            pltpu.VMEM((64, _D), jnp.float32),
            pltpu.VMEM_SHARED((_NS * 2 * (_CA + 8), _D), jnp.float32),
            pltpu.SemaphoreType.DMA, pltpu.SemaphoreType.DMA,
            pltpu.SemaphoreType.DMA, pltpu.SemaphoreType.DMA,
            pltpu.SemaphoreType.DMA, pltpu.SemaphoreType.DMA,
            pltpu.SemaphoreType.DMA, pltpu.SemaphoreType.DMA,
            pltpu.SemaphoreType.DMA, pltpu.SemaphoreType.DMA,
            pltpu.SemaphoreType.DMA,
        ],
    )(_z0_body)


def _z0_body(table_hbm, pidx_hbm, pb_hbm, pw_hbm, starts_hbm, z0_hbm,
             starts_vbuf, wv0, wv1, wv2, ix0, ix1, ix2, pb0, pb1, pb2,
             sg0, sg1, sg2, rb0, rb1, rb2, zbuf, acc,
             lsem0, lsem1, lsem2, gsem0, gsem1, gsem2,
             ssem0, ssem1, ssem2, fsem0, fsem1):
    s = lax.axis_index("s")
    c = lax.axis_index("c")
    tid = s * _NC + c
    wv = (wv0, wv1, wv2)
    ix = (ix0, ix1, ix2)
    pbb = (pb0, pb1, pb2)
    sg = (sg0, sg1, sg2)
    rb = (rb0, rb1, rb2)
    lsem = (lsem0, lsem1, lsem2)
    gsem = (gsem0, gsem1, gsem2)
    ssem = (ssem0, ssem1, ssem2)
    fsem = (fsem0, fsem1)

    ci0 = tid * _NKA
    ci0a = pl.multiple_of(ci0, 8)
    pltpu.sync_copy(starts_hbm.at[pl.ds(ci0a, 56)], starts_vbuf)
    _zero_rows(zbuf, 64)
    lane = jnp.arange(16, dtype=jnp.int32)

    def read_start(ci):
        rel = ci - ci0
        cb8 = pl.multiple_of((rel // 8) * 8, 8)
        sv = starts_vbuf[pl.ds(cb8, 16)]
        r = rel - cb8
        return sv.at[(lane + r) & 15].get(mode="promise_in_bounds")[0]

    reg0 = s * (2 * (_CA + 8))

    for r in (0, 1):
        for k in (0, 64, 128, 192):
            pltpu.sync_copy(zbuf,
                            acc.at[pl.ds(reg0 + r * (_CA + 8) + k, 64)])

    def start_lin(p, b):
        pltpu.async_copy(pidx_hbm.at[pl.ds(p, 128)], ix[b], lsem[b])
        pltpu.async_copy(pb_hbm.at[pl.ds(p, 128)], pbb[b], lsem[b])
        pltpu.async_copy(pw_hbm.at[pl.ds(p, 128)], wv[b], lsem[b])

    def wait_lin(p, b):
        pltpu.make_async_copy(pidx_hbm.at[pl.ds(p, 128)], ix[b],
                              lsem[b]).wait()
        pltpu.make_async_copy(pb_hbm.at[pl.ds(p, 128)], pbb[b],
                              lsem[b]).wait()
        pltpu.make_async_copy(pw_hbm.at[pl.ds(p, 128)], wv[b], lsem[b]).wait()

    @pl.loop(ci0, ci0 + _NKA, step=2)
    def chunk_pair(cb):
        for r in (0, 1):
            ci = cb + r
            base_e = ci * _CA
            rbase = reg0 + r * (_CA + 8)
            cs = read_start(ci)
            ce = read_start(ci + 1)

            @pl.when((ci >= ci0 + 2) & ((ci - 2) * _CA < _E))
            def _flushprev():
                pltpu.async_copy(
                    acc.at[pl.ds(rbase, _CA)],
                    z0_hbm.at[pl.ds((ci - 2) * _CA, _CA)], fsem[r]).wait()
                for k in (0, 64, 128, 192):
                    pltpu.sync_copy(zbuf, acc.at[pl.ds(rbase + k, 64)])

            p0 = pl.multiple_of((cs // 128) * 128, 128)
            ns = (ce - p0 + 127) // 128

            @pl.when(ns >= 1)
            def _prol0():
                start_lin(p0, 0)
                wait_lin(p0, 0)
                pltpu.async_copy(table_hbm.at[ix[0]], rb[0], gsem[0])

            @pl.when(ns >= 2)
            def _prol1():
                start_lin(p0 + 128, 1)

            @pl.loop(0, ns, step=3)
            def subtri(k):
                for b in (0, 1, 2):
                    kk = k + b
                    bn = (b + 1) % 3
                    bp = (b + 2) % 3

                    @pl.when(kk < ns)
                    def _sub():
                        p = pl.multiple_of(p0 + 128 * kk, 128)

                        @pl.when(kk + 1 < ns)
                        def _nextg():
                            wait_lin(p + 128, bn)

                            @pl.when(kk >= 2)
                            def _drainbn():
                                pltpu.make_async_copy(
                                    rb[bn], acc.at[sg[bn]], ssem[bn]).wait()

                            pltpu.async_copy(table_hbm.at[ix[bn]], rb[bn],
                                             gsem[bn])

                        @pl.when(kk + 2 < ns)
                        def _nextl():
                            start_lin(p + 256, bp)

                        for j in range(8):
                            pbv = pbb[b][pl.ds(16 * j, 16)]
                            g = p + 16 * j + lane
                            live = (g >= cs) & (g < ce)
                            seg = pbv - base_e
                            seg = jnp.maximum(seg, 0)
                            seg = jnp.minimum(seg, _CA - 1)
                            seg = jnp.where(live, seg, _CA)
                            sg[b][pl.ds(16 * j, 16)] = seg + rbase
                            wvv = wv[b][pl.ds(16 * j, 16)]
                            wvv = jnp.where(live, wvv, jnp.float32(0.0))
                            wv[b][pl.ds(16 * j, 16)] = wvv

                        pltpu.make_async_copy(table_hbm.at[ix[b]], rb[b],
                                              gsem[b]).wait()

                        def sgrp(jj, carry):
                            base_i = pl.multiple_of(16 * jj, 16)
                            wvec = wv[b][pl.ds(base_i, 16)]
                            for l in range(16):
                                i = base_i + l
                                w = wvec[l]
                                for j in range(8):
                                    sl = pl.ds(16 * j, 16)
                                    rb[b][i, sl] = rb[b][i, sl] * w
                            return carry

                        lax.fori_loop(0, 8, sgrp, 0)
                        pltpu.async_copy(rb[b], acc.at[sg[b]], ssem[b],
                                         add=True)

            for b in (0, 1, 2):
                @pl.when(ns > b)
                def _draintail():
                    pltpu.make_async_copy(rb[b], acc.at[sg[b]],
                                          ssem[b]).wait()

    for r in (0, 1):
        ci_last = ci0 + _NKA - 2 + r

        @pl.when(ci_last * _CA < _E)
        def _fdrain():
            pltpu.async_copy(
                acc.at[pl.ds(reg0 + r * (_CA + 8), _CA)],
                z0_hbm.at[pl.ds(ci_last * _CA, _CA)], fsem[r]).wait()


@functools.lru_cache(maxsize=None)
def _build_msg_kernel():
    return functools.partial(
        pl.kernel,
        out_type=jax.ShapeDtypeStruct((_NC, _N, _D), jnp.float32),
        mesh=_sc_mesh(),
        scratch_types=[
            pltpu.VMEM((_CB,), jnp.int32), pltpu.VMEM((_CB,), jnp.int32),
            pltpu.VMEM((_CB,), jnp.int32), pltpu.VMEM((_CB,), jnp.int32),
            pltpu.VMEM((_CB,), jnp.int32), pltpu.VMEM((_CB,), jnp.int32),
            pltpu.VMEM((_CB, _D), jnp.float32),
            pltpu.VMEM((_CB, _D), jnp.float32),
            pltpu.VMEM((_CB, _D), jnp.float32),
            pltpu.VMEM((_CB, _D), jnp.float32),
            pltpu.VMEM((_CB, _D), jnp.float32),
            pltpu.VMEM((_CB, _D), jnp.float32),
            pltpu.VMEM_SHARED((_N, _D), jnp.float32),
            pltpu.SemaphoreType.DMA, pltpu.SemaphoreType.DMA,
            pltpu.SemaphoreType.DMA, pltpu.SemaphoreType.DMA,
            pltpu.SemaphoreType.DMA, pltpu.SemaphoreType.DMA,
            pltpu.SemaphoreType.DMA, pltpu.SemaphoreType.DMA,
            pltpu.SemaphoreType.DMA,
        ],
    )(_msg_body)


def _msg_body(emb_hbm, h_hbm, src_hbm, dst_hbm, out_hbm,
              sb0, sb1, sb2, db0, db1, db2, eb0, eb1, eb2, hb0, hb1, hb2,
              acc, lsem0, lsem1, lsem2, gsem0, gsem1, gsem2,
              csem0, csem1, csem2):
    s = lax.axis_index("s")
    c = lax.axis_index("c")
    tid = s * _NC + c
    sb = (sb0, sb1, sb2)
    db = (db0, db1, db2)
    eb = (eb0, eb1, eb2)
    hb = (hb0, hb1, hb2)
    lsem = (lsem0, lsem1, lsem2)
    gsem = (gsem0, gsem1, gsem2)
    csem = (csem0, csem1, csem2)

    _zero_rows(eb0, _CB)
    row0 = s * _ROWS_PT
    for k in range(0, 576, 64):
        pltpu.sync_copy(eb0, acc.at[pl.ds(row0 + k, 64)])
    pltpu.sync_copy(eb0.at[pl.ds(0, 49)], acc.at[pl.ds(row0 + 576, 49)])
    plsc.subcore_barrier()

    def bases(kk):
        base = (tid + _NW * kk) * _CB
        bemb = jnp.minimum(base, _E - _CB)
        return base, bemb

    def start_lin(kk, b):
        base, bemb = bases(kk)
        pltpu.async_copy(emb_hbm.at[pl.ds(bemb, _CB)], eb[b], lsem[b])
        pltpu.async_copy(src_hbm.at[pl.ds(base, _CB)], sb[b], lsem[b])
        pltpu.async_copy(dst_hbm.at[pl.ds(base, _CB)], db[b], lsem[b])

    def wait_lin(kk, b):
        base, bemb = bases(kk)
        pltpu.make_async_copy(emb_hbm.at[pl.ds(bemb, _CB)], eb[b],
                              lsem[b]).wait()
        pltpu.make_async_copy(src_hbm.at[pl.ds(base, _CB)], sb[b],
                              lsem[b]).wait()
        pltpu.make_async_copy(dst_hbm.at[pl.ds(base, _CB)], db[b],
                              lsem[b]).wait()

    start_lin(0, 0)
    wait_lin(0, 0)
    pltpu.async_copy(h_hbm.at[sb[0]], hb[0], gsem[0])
    start_lin(1, 1)

    @pl.loop(0, _NK, step=3)
    def tri(k):
        for b in (0, 1, 2):
            kk = k + b
            bn = (b + 1) % 3
            bp = (b + 2) % 3

            @pl.when(kk < _NK)
            def _chunk():
                @pl.when(kk + 1 < _NK)
                def _nextg():
                    wait_lin(kk + 1, bn)

                    @pl.when(kk >= 2)
                    def _drainbn():
                        pltpu.make_async_copy(hb[bn], acc.at[db[bn]],
                                              csem[bn]).wait()

                    pltpu.async_copy(h_hbm.at[sb[bn]], hb[bn], gsem[bn])

                @pl.when(kk + 2 < _NK)
                def _nextl():
                    start_lin(kk + 2, bp)

                base, _ = bases(kk)
                nvalid = jnp.minimum(jnp.maximum(_E - base, 0), _CB)
                pltpu.make_async_copy(h_hbm.at[sb[b]], hb[b],
                                      gsem[b]).wait()

                def row(i, carry):
                    live = i < nvalid
                    for j in range(8):
                        sl = pl.ds(16 * j, 16)
                        m = jnp.maximum(eb[b][i, sl] + hb[b][i, sl],
                                        jnp.float32(0.0))
                        hb[b][i, sl] = jnp.where(live, m, jnp.float32(0.0))
                    return carry

                lax.fori_loop(0, _CB, row, 0)
                pltpu.async_copy(hb[b], acc.at[db[b]], csem[b], add=True)

    for b in (0, 1, 2):
        m = _NK - 1 - ((_NK - 1 - b) % 3)
        if m >= 0:
            pltpu.make_async_copy(hb[b], acc.at[db[b]], csem[b]).wait()

    plsc.subcore_barrier()
    orow0 = s * 624
    for k, cnt in ((0, 128), (128, 128), (256, 128), (384, 128), (496, 128)):
        pltpu.sync_copy(acc.at[pl.ds(orow0 + k, cnt)],
                        out_hbm.at[c, pl.ds(orow0 + k, cnt)])

    @pl.when(s == _NS - 1)
    def _tail():
        pltpu.sync_copy(acc.at[pl.ds(9984, 16)],
                        out_hbm.at[c, pl.ds(9984, 16)])


_EB = 1000


def _edge_emb_body(z0_ref, ea_ref, a1_ref, b1_ref, wl_ref, a2_ref, b2_ref,
                   we_ref, wz_ref, bb_ref, o0_ref, o1_ref, o2_ref):
    zz = jnp.maximum(z0_ref[:] * a1_ref[:] + b1_ref[:], 0.0)
    zz = jnp.dot(zz, wl_ref[:], preferred_element_type=jnp.float32)
    zz = jnp.maximum(zz * a2_ref[:] + b2_ref[:], 0.0)
    ea = ea_ref[:]
    outs = (o0_ref, o1_ref, o2_ref)
    for l in range(_L):
        o = jnp.dot(ea, we_ref[l], preferred_element_type=jnp.float32)
        o = o + jnp.dot(zz, wz_ref[l], preferred_element_type=jnp.float32)
        outs[l][:] = o + bb_ref[pl.ds(l, 1), :]


def _edge_emb_call(z0, ea_pad, a1, b1, wl, a2, b2, we_pad, wz, bb):
    n_blk = _E // _EB
    full2 = lambda shape: pl.BlockSpec(shape, lambda i: (0, 0))
    full3 = lambda shape: pl.BlockSpec(shape, lambda i: (0, 0, 0))
    row = pl.BlockSpec((_EB, _D), lambda i: (i, 0))
    out_sh = jax.ShapeDtypeStruct((_E, _D), jnp.float32)
    return pl.pallas_call(
        _edge_emb_body,
        grid=(n_blk,),
        in_specs=[
            row,
            pl.BlockSpec((_EB, 8), lambda i: (i, 0)),
            full2((1, _D)), full2((1, _D)),
            full2((_D, _D)),
            full2((1, _D)), full2((1, _D)),
            full3((_L, 8, _D)),
            full3((_L, _D, _D)),
            full2((_L, _D)),
        ],
        out_specs=[row, row, row],
        out_shape=[out_sh, out_sh, out_sh],
    )(z0, ea_pad, a1, b1, wl, a2, b2, we_pad, wz, bb)


_ND = 1000


def _node_body(do_relu, h_ref, p0_ref, p1_ref, ev_ref, w1_ref, c1_ref,
               w2_ref, c2_ref, out_ref):
    t = h_ref[:] * ev_ref[:] + p0_ref[:] + p1_ref[:]
    u = jnp.dot(t, w1_ref[:], preferred_element_type=jnp.float32) + c1_ref[:]
    u = jnp.maximum(u, 0.0)
    v = jnp.dot(u, w2_ref[:], preferred_element_type=jnp.float32) + c2_ref[:]
    if do_relu:
        v = jnp.maximum(v, 0.0)
    out_ref[:] = v


def _node_call(do_relu, h, part0, part1, epsv, w1f, c1f, w2f, c2f):
    n_blk = _N // _ND
    row = pl.BlockSpec((_ND, _D), lambda i: (i, 0))
    full2 = lambda shape: pl.BlockSpec(shape, lambda i: (0, 0))
    return pl.pallas_call(
        functools.partial(_node_body, do_relu),
        grid=(n_blk,),
        in_specs=[
            row, row, row,
            full2((1, _D)),
            full2((_D, 2 * _D)), full2((1, 2 * _D)),
            full2((2 * _D, _D)), full2((1, _D)),
        ],
        out_specs=row,
        out_shape=jax.ShapeDtypeStruct((_N, _D), jnp.float32),
    )(h, part0, part1, epsv, w1f, c1f, w2f, c2f)


def kernel(x, edge_index, edge_attr, batch, pos_index, pos_enc, pos_batch,
           z_initial_weight, z_lin_W, z_lin_b, z_bn1_g, z_bn1_b, z_bn2_g,
           z_bn2_b, edge_enc_W, edge_enc_b, edge_pos_W, edge_pos_b, eps_param,
           mlp_W1, mlp_b1, mlp_bn_g, mlp_bn_b, mlp_W2, mlp_b2, out_bn_g,
           out_bn_b):
    f32 = jnp.float32
    inv = jnp.asarray(1.0 / jnp.sqrt(1.0 + 1e-5), f32)

    starts = jnp.searchsorted(
        pos_batch,
        jnp.arange(0, _NCA + 1, dtype=jnp.int32) * _CA).astype(jnp.int32)
    starts = jnp.concatenate(
        [starts, jnp.zeros((_NSTARTS - _NCA - 1,), jnp.int32)])

    z0 = _build_z0_kernel()(z_initial_weight, pos_index, pos_batch, pos_enc,
                            starts)

    a1 = (z_bn1_g * inv).reshape(1, _D)
    b1 = z_bn1_b.reshape(1, _D)
    a2 = (z_bn2_g * inv).reshape(1, _D)
    b2 = (z_bn2_b + z_lin_b * z_bn2_g * inv).reshape(1, _D)

    ea_pad = jnp.pad(edge_attr, ((0, 0), (0, 1)))
    we_pad = jnp.pad(edge_enc_W, ((0, 0), (0, 1), (0, 0)))
    bb = edge_enc_b + edge_pos_b
    emb = _edge_emb_call(z0, ea_pad, a1, b1, z_lin_W, a2, b2, we_pad,
                         edge_pos_W, bb)

    src = jnp.pad(edge_index[0], (0, _PADE - _E))
    dst = jnp.pad(edge_index[1], (0, _PADE - _E))
    h = x
    for l in range(_L):
        parts = _build_msg_kernel()(emb[l], h, src, dst)
        ga = mlp_bn_g[l] * inv
        w1f = mlp_W1[l] * ga[None, :]
        c1f = (mlp_b1[l] * ga + mlp_bn_b[l]).reshape(1, 2 * _D)
        go = out_bn_g[l] * inv
        w2f = mlp_W2[l] * go[None, :]
        c2f = (mlp_b2[l] * go + out_bn_b[l]).reshape(1, _D)
        epsv = jnp.full((1, _D), 1.0 + eps_param[l], f32)
        h = _node_call(l < _L - 1, h, parts[0], parts[1], epsv, w1f, c1f,
                       w2f, c2f)
    return h


# speedup vs baseline: 1.1922x; 1.1922x over previous
import functools

import jax
import jax.numpy as jnp
from jax import lax
from jax.experimental import pallas as pl
from jax.experimental.pallas import tpu as pltpu
from jax.experimental.pallas import tpu_sc as plsc

_N = 10000
_E = 320000
_P = 1280000
_D = 128
_L = 3
_Z = 1800

_NC = 2
_NS = 16
_NW = _NC * _NS

_CA = 256
_NCA = 1280
_NKA = _NCA // _NW
_NSTARTS = 1336
_CB = 128
_NCHUNK_C = _E // _CB
_ROWS_PT = _N // _NS

def _sc_mesh():
    return plsc.VectorSubcoreMesh(core_axis_name="c", subcore_axis_name="s",
                                  num_cores=_NC, num_subcores=_NS)


def _zero_rows(buf, nrows):
    zv = jnp.zeros((16,), jnp.float32)

    def body(i, carry):
        for j in range(_D // 16):
            buf[i, pl.ds(16 * j, 16)] = zv
        return carry

    lax.fori_loop(0, nrows, body, 0)


@functools.lru_cache(maxsize=None)
def _build_z0_kernel():
    return functools.partial(
        pl.kernel,
        out_type=jax.ShapeDtypeStruct((_E, _D), jnp.float32),
        mesh=_sc_mesh(),
        scratch_types=[
            pltpu.VMEM((56,), jnp.int32),
            pltpu.VMEM((128,), jnp.float32), pltpu.VMEM((128,), jnp.float32),
            pltpu.VMEM((128,), jnp.float32),
            pltpu.VMEM((128,), jnp.int32), pltpu.VMEM((128,), jnp.int32),
            pltpu.VMEM((128,), jnp.int32),
            pltpu.VMEM((128,), jnp.int32), pltpu.VMEM((128,), jnp.int32),
            pltpu.VMEM((128,), jnp.int32),
            pltpu.VMEM((128,), jnp.int32), pltpu.VMEM((128,), jnp.int32),
            pltpu.VMEM((128,), jnp.int32),
            pltpu.VMEM((128, _D), jnp.float32),
            pltpu.VMEM((128, _D), jnp.float32),
            pltpu.VMEM((128, _D), jnp.float32),
            pltpu.VMEM((64, _D), jnp.float32),
            pltpu.VMEM_SHARED((_NS * 2 * (_CA + 8), _D), jnp.float32),
            pltpu.SemaphoreType.DMA, pltpu.SemaphoreType.DMA,
            pltpu.SemaphoreType.DMA, pltpu.SemaphoreType.DMA,
            pltpu.SemaphoreType.DMA, pltpu.SemaphoreType.DMA,
            pltpu.SemaphoreType.DMA, pltpu.SemaphoreType.DMA,
            pltpu.SemaphoreType.DMA, pltpu.SemaphoreType.DMA,
            pltpu.SemaphoreType.DMA,
        ],
    )(_z0_body)


def _z0_body(table_hbm, pidx_hbm, pb_hbm, pw_hbm, starts_hbm, z0_hbm,
             starts_vbuf, wv0, wv1, wv2, ix0, ix1, ix2, pb0, pb1, pb2,
             sg0, sg1, sg2, rb0, rb1, rb2, zbuf, acc,
             lsem0, lsem1, lsem2, gsem0, gsem1, gsem2,
             ssem0, ssem1, ssem2, fsem0, fsem1):
    s = lax.axis_index("s")
    c = lax.axis_index("c")
    tid = s * _NC + c
    wv = (wv0, wv1, wv2)
    ix = (ix0, ix1, ix2)
    pbb = (pb0, pb1, pb2)
    sg = (sg0, sg1, sg2)
    rb = (rb0, rb1, rb2)
    lsem = (lsem0, lsem1, lsem2)
    gsem = (gsem0, gsem1, gsem2)
    ssem = (ssem0, ssem1, ssem2)
    fsem = (fsem0, fsem1)

    ci0 = tid * _NKA
    ci0a = pl.multiple_of(ci0, 8)
    pltpu.sync_copy(starts_hbm.at[pl.ds(ci0a, 56)], starts_vbuf)
    _zero_rows(zbuf, 64)
    lane = jnp.arange(16, dtype=jnp.int32)

    def read_start(ci):
        rel = ci - ci0
        cb8 = pl.multiple_of((rel // 8) * 8, 8)
        sv = starts_vbuf[pl.ds(cb8, 16)]
        r = rel - cb8
        return sv.at[(lane + r) & 15].get(mode="promise_in_bounds")[0]

    reg0 = s * (2 * (_CA + 8))

    for r in (0, 1):
        for k in (0, 64, 128, 192):
            pltpu.sync_copy(zbuf,
                            acc.at[pl.ds(reg0 + r * (_CA + 8) + k, 64)])

    def start_lin(p, b):
        pltpu.async_copy(pidx_hbm.at[pl.ds(p, 128)], ix[b], lsem[b])
        pltpu.async_copy(pb_hbm.at[pl.ds(p, 128)], pbb[b], lsem[b])
        pltpu.async_copy(pw_hbm.at[pl.ds(p, 128)], wv[b], lsem[b])

    def wait_lin(p, b):
        pltpu.make_async_copy(pidx_hbm.at[pl.ds(p, 128)], ix[b],
                              lsem[b]).wait()
        pltpu.make_async_copy(pb_hbm.at[pl.ds(p, 128)], pbb[b],
                              lsem[b]).wait()
        pltpu.make_async_copy(pw_hbm.at[pl.ds(p, 128)], wv[b], lsem[b]).wait()

    @pl.loop(ci0, ci0 + _NKA, step=2)
    def chunk_pair(cb):
        for r in (0, 1):
            ci = cb + r
            base_e = ci * _CA
            rbase = reg0 + r * (_CA + 8)
            cs = read_start(ci)
            ce = read_start(ci + 1)

            @pl.when((ci >= ci0 + 2) & ((ci - 2) * _CA < _E))
            def _flushprev():
                pltpu.async_copy(
                    acc.at[pl.ds(rbase, _CA)],
                    z0_hbm.at[pl.ds((ci - 2) * _CA, _CA)], fsem[r]).wait()
                for k in (0, 64, 128, 192):
                    pltpu.sync_copy(zbuf, acc.at[pl.ds(rbase + k, 64)])

            p0 = pl.multiple_of((cs // 128) * 128, 128)
            ns = (ce - p0 + 127) // 128

            @pl.when(ns >= 1)
            def _prol0():
                start_lin(p0, 0)
                wait_lin(p0, 0)
                pltpu.async_copy(table_hbm.at[ix[0]], rb[0], gsem[0])

            @pl.when(ns >= 2)
            def _prol1():
                start_lin(p0 + 128, 1)

            @pl.loop(0, ns, step=3)
            def subtri(k):
                for b in (0, 1, 2):
                    kk = k + b
                    bn = (b + 1) % 3
                    bp = (b + 2) % 3

                    @pl.when(kk < ns)
                    def _sub():
                        p = pl.multiple_of(p0 + 128 * kk, 128)

                        @pl.when(kk + 1 < ns)
                        def _nextg():
                            wait_lin(p + 128, bn)

                            @pl.when(kk >= 2)
                            def _drainbn():
                                pltpu.make_async_copy(
                                    rb[bn], acc.at[sg[bn]], ssem[bn]).wait()

                            pltpu.async_copy(table_hbm.at[ix[bn]], rb[bn],
                                             gsem[bn])

                        @pl.when(kk + 2 < ns)
                        def _nextl():
                            start_lin(p + 256, bp)

                        for j in range(8):
                            pbv = pbb[b][pl.ds(16 * j, 16)]
                            g = p + 16 * j + lane
                            live = (g >= cs) & (g < ce)
                            seg = pbv - base_e
                            seg = jnp.maximum(seg, 0)
                            seg = jnp.minimum(seg, _CA - 1)
                            seg = jnp.where(live, seg, _CA)
                            sg[b][pl.ds(16 * j, 16)] = seg + rbase
                            wvv = wv[b][pl.ds(16 * j, 16)]
                            wvv = jnp.where(live, wvv, jnp.float32(0.0))
                            wv[b][pl.ds(16 * j, 16)] = wvv

                        pltpu.make_async_copy(table_hbm.at[ix[b]], rb[b],
                                              gsem[b]).wait()

                        def sgrp(jj, carry):
                            base_i = pl.multiple_of(16 * jj, 16)
                            wvec = wv[b][pl.ds(base_i, 16)]
                            for l in range(16):
                                i = base_i + l
                                w = wvec[l]
                                for j in range(8):
                                    sl = pl.ds(16 * j, 16)
                                    rb[b][i, sl] = rb[b][i, sl] * w
                            return carry

                        lax.fori_loop(0, 8, sgrp, 0)
                        pltpu.async_copy(rb[b], acc.at[sg[b]], ssem[b],
                                         add=True)

            for b in (0, 1, 2):
                @pl.when(ns > b)
                def _draintail():
                    pltpu.make_async_copy(rb[b], acc.at[sg[b]],
                                          ssem[b]).wait()

    for r in (0, 1):
        ci_last = ci0 + _NKA - 2 + r

        @pl.when(ci_last * _CA < _E)
        def _fdrain():
            pltpu.async_copy(
                acc.at[pl.ds(reg0 + r * (_CA + 8), _CA)],
                z0_hbm.at[pl.ds(ci_last * _CA, _CA)], fsem[r]).wait()


@functools.lru_cache(maxsize=None)
def _build_msg_kernel():
    return functools.partial(
        pl.kernel,
        out_type=jax.ShapeDtypeStruct((_NC, _N, _D), jnp.float32),
        mesh=_sc_mesh(),
        scratch_types=[
            pltpu.VMEM((_CB,), jnp.int32),
            pltpu.VMEM((_CB,), jnp.int32),
            pltpu.VMEM((_CB, _D), jnp.float32),
            pltpu.VMEM((_CB, _D), jnp.float32),
            pltpu.VMEM((128, _D), jnp.float32),
            pltpu.VMEM_SHARED((_N, _D), jnp.float32),
            pltpu.SemaphoreType.DMA,
            pltpu.SemaphoreType.DMA,
        ],
    )(_msg_body)


def _msg_body(emb_hbm, h_hbm, src_hbm, dst_hbm, out_hbm,
              sbuf, dbuf, embuf, hgbuf, zbuf, acc, sem, sem2):
    s = lax.axis_index("s")
    c = lax.axis_index("c")
    tid = s * _NC + c
    _zero_rows(zbuf, 128)
    row0 = s * _ROWS_PT
    for k, cnt in ((0, 128), (128, 128), (256, 128), (384, 128), (512, 113)):
        pltpu.sync_copy(zbuf.at[pl.ds(0, cnt)], acc.at[pl.ds(row0 + k, cnt)])
    plsc.subcore_barrier()

    @pl.loop(tid, _NCHUNK_C, step=_NW)
    def do_chunk(ci):
        base = ci * _CB
        pltpu.sync_copy(emb_hbm.at[pl.ds(base, _CB)], embuf)
        pltpu.sync_copy(src_hbm.at[pl.ds(base, _CB)], sbuf)
        pltpu.sync_copy(dst_hbm.at[pl.ds(base, _CB)], dbuf)
        pltpu.async_copy(h_hbm.at[sbuf], hgbuf, sem).wait()

        def row(i, carry):
            for j in range(8):
                sl = pl.ds(16 * j, 16)
                m = embuf[i, sl] + hgbuf[i, sl]
                embuf[i, sl] = jnp.maximum(m, jnp.float32(0.0))
            return carry

        lax.fori_loop(0, _CB, row, 0)
        pltpu.async_copy(embuf, acc.at[dbuf], sem2, add=True).wait()

    plsc.subcore_barrier()
    orow0 = s * 624
    for k, cnt in ((0, 128), (128, 128), (256, 128), (384, 128), (496, 128)):
        pltpu.sync_copy(acc.at[pl.ds(orow0 + k, cnt)],
                        out_hbm.at[c, pl.ds(orow0 + k, cnt)])

    @pl.when(s == _NS - 1)
    def _tail():
        pltpu.sync_copy(acc.at[pl.ds(9984, 16)],
                        out_hbm.at[c, pl.ds(9984, 16)])


_EB = 1000


def _edge_emb_body(z0_ref, ea_ref, a1_ref, b1_ref, wl_ref, a2_ref, b2_ref,
                   we_ref, wz_ref, bb_ref, o0_ref, o1_ref, o2_ref):
    zz = jnp.maximum(z0_ref[:] * a1_ref[:] + b1_ref[:], 0.0)
    zz = jnp.dot(zz, wl_ref[:], preferred_element_type=jnp.float32)
    zz = jnp.maximum(zz * a2_ref[:] + b2_ref[:], 0.0)
    ea = ea_ref[:]
    outs = (o0_ref, o1_ref, o2_ref)
    for l in range(_L):
        o = jnp.dot(ea, we_ref[l], preferred_element_type=jnp.float32)
        o = o + jnp.dot(zz, wz_ref[l], preferred_element_type=jnp.float32)
        outs[l][:] = o + bb_ref[pl.ds(l, 1), :]


def _edge_emb_call(z0, ea_pad, a1, b1, wl, a2, b2, we_pad, wz, bb):
    n_blk = _E // _EB
    full2 = lambda shape: pl.BlockSpec(shape, lambda i: (0, 0))
    full3 = lambda shape: pl.BlockSpec(shape, lambda i: (0, 0, 0))
    row = pl.BlockSpec((_EB, _D), lambda i: (i, 0))
    out_sh = jax.ShapeDtypeStruct((_E, _D), jnp.float32)
    return pl.pallas_call(
        _edge_emb_body,
        grid=(n_blk,),
        in_specs=[
            row,
            pl.BlockSpec((_EB, 8), lambda i: (i, 0)),
            full2((1, _D)), full2((1, _D)),
            full2((_D, _D)),
            full2((1, _D)), full2((1, _D)),
            full3((_L, 8, _D)),
            full3((_L, _D, _D)),
            full2((_L, _D)),
        ],
        out_specs=[row, row, row],
        out_shape=[out_sh, out_sh, out_sh],
    )(z0, ea_pad, a1, b1, wl, a2, b2, we_pad, wz, bb)


_ND = 1000


def _node_body(do_relu, h_ref, p0_ref, p1_ref, ev_ref, w1_ref, c1_ref,
               w2_ref, c2_ref, out_ref):
    t = h_ref[:] * ev_ref[:] + p0_ref[:] + p1_ref[:]
    u = jnp.dot(t, w1_ref[:], preferred_element_type=jnp.float32) + c1_ref[:]
    u = jnp.maximum(u, 0.0)
    v = jnp.dot(u, w2_ref[:], preferred_element_type=jnp.float32) + c2_ref[:]
    if do_relu:
        v = jnp.maximum(v, 0.0)
    out_ref[:] = v


def _node_call(do_relu, h, part0, part1, epsv, w1f, c1f, w2f, c2f):
    n_blk = _N // _ND
    row = pl.BlockSpec((_ND, _D), lambda i: (i, 0))
    full2 = lambda shape: pl.BlockSpec(shape, lambda i: (0, 0))
    return pl.pallas_call(
        functools.partial(_node_body, do_relu),
        grid=(n_blk,),
        in_specs=[
            row, row, row,
            full2((1, _D)),
            full2((_D, 2 * _D)), full2((1, 2 * _D)),
            full2((2 * _D, _D)), full2((1, _D)),
        ],
        out_specs=row,
        out_shape=jax.ShapeDtypeStruct((_N, _D), jnp.float32),
    )(h, part0, part1, epsv, w1f, c1f, w2f, c2f)


def kernel(x, edge_index, edge_attr, batch, pos_index, pos_enc, pos_batch,
           z_initial_weight, z_lin_W, z_lin_b, z_bn1_g, z_bn1_b, z_bn2_g,
           z_bn2_b, edge_enc_W, edge_enc_b, edge_pos_W, edge_pos_b, eps_param,
           mlp_W1, mlp_b1, mlp_bn_g, mlp_bn_b, mlp_W2, mlp_b2, out_bn_g,
           out_bn_b):
    f32 = jnp.float32
    inv = jnp.asarray(1.0 / jnp.sqrt(1.0 + 1e-5), f32)

    starts = jnp.searchsorted(
        pos_batch,
        jnp.arange(0, _NCA + 1, dtype=jnp.int32) * _CA).astype(jnp.int32)
    starts = jnp.concatenate(
        [starts, jnp.zeros((_NSTARTS - _NCA - 1,), jnp.int32)])

    z0 = _build_z0_kernel()(z_initial_weight, pos_index, pos_batch, pos_enc,
                            starts)

    a1 = (z_bn1_g * inv).reshape(1, _D)
    b1 = z_bn1_b.reshape(1, _D)
    a2 = (z_bn2_g * inv).reshape(1, _D)
    b2 = (z_bn2_b + z_lin_b * z_bn2_g * inv).reshape(1, _D)

    ea_pad = jnp.pad(edge_attr, ((0, 0), (0, 1)))
    we_pad = jnp.pad(edge_enc_W, ((0, 0), (0, 1), (0, 0)))
    bb = edge_enc_b + edge_pos_b
    emb = _edge_emb_call(z0, ea_pad, a1, b1, z_lin_W, a2, b2, we_pad,
                         edge_pos_W, bb)

    src = edge_index[0]
    dst = edge_index[1]
    h = x
    for l in range(_L):
        parts = _build_msg_kernel()(emb[l], h, src, dst)
        ga = mlp_bn_g[l] * inv
        w1f = mlp_W1[l] * ga[None, :]
        c1f = (mlp_b1[l] * ga + mlp_bn_b[l]).reshape(1, 2 * _D)
        go = out_bn_g[l] * inv
        w2f = mlp_W2[l] * go[None, :]
        c2f = (mlp_b2[l] * go + out_bn_b[l]).reshape(1, _D)
        epsv = jnp.full((1, _D), 1.0 + eps_param[l], f32)
        h = _node_call(l < _L - 1, h, parts[0], parts[1], epsv, w1f, c1f,
                       w2f, c2f)
    return h
